# Initial kernel scaffold; baseline (speedup 1.0000x reference)
import jax
import jax.numpy as jnp
from jax.experimental import pallas as pl


def kernel(t, x, W_season, b_season, W_in, b_in, W_self0, W_msg0, b0, W_self1, W_msg1, b1, W_out, b_out, W_e1, b_e1, W_e2, b_e2, edge_index):
    raise NotImplementedError("write your pallas kernel here")


# speedup vs baseline: 440.3562x; 440.3562x over previous
import jax
import jax.numpy as jnp
import numpy as np
from jax.experimental import pallas as pl
from jax.experimental.pallas import tpu as pltpu

_TWO_PI = 2.0 * np.pi
_S = 32
_H = 128


def _fused_body(t_ref, x_ref, ws_ref, bs_ref, winx_ref, wins_ref, bin_ref,
                w0_ref, b0_ref, w1_ref, b1_ref, wout_ref, bout_ref,
                we1_ref, be1_ref, we2_ref, be2_ref, out_ref):
    B = x_ref.shape[0]
    tv = t_ref[:]
    st = jnp.sin(_TWO_PI * tv)
    ct = jnp.cos(_TWO_PI * tv)
    s_emb = st * ws_ref[0:1, :] + ct * ws_ref[1:2, :] + bs_ref[:]
    bias_eff = jnp.dot(s_emb, wins_ref[:],
                       preferred_element_type=jnp.float32) + bin_ref[:]
    x = x_ref[:]
    p = jnp.dot(x, winx_ref[:], preferred_element_type=jnp.float32) + bias_eff
    h1 = jnp.tanh(jnp.dot(p, w0_ref[:], preferred_element_type=jnp.float32)
                  + b0_ref[:])
    h2 = jnp.dot(h1, w1_ref[:], preferred_element_type=jnp.float32) + b1_ref[:]
    d = jnp.dot(h2, wout_ref[:], preferred_element_type=jnp.float32) + bout_ref[:]
    c = jnp.mean(d, axis=1, keepdims=True)
    e1 = jnp.tanh(jnp.dot(x, we1_ref[:], preferred_element_type=jnp.float32)
                  + be1_ref[:])
    e = jnp.dot(e1, we2_ref[:], preferred_element_type=jnp.float32) + be2_ref[:]
    out_ref[:] = jnp.broadcast_to(c, (B, _S)) + e


def kernel(t, x, W_season, b_season, W_in, b_in, W_self0, W_msg0, b0,
           W_self1, W_msg1, b1, W_out, b_out, W_e1, b_e1, W_e2, b_e2,
           edge_index):
    B = x.shape[0]
    f32 = jnp.float32
    t2 = t.reshape(1, 1)
    winx = W_in[:_S, :]
    wins = W_in[_S:, :]
    we1p = jnp.zeros((_S, 32), f32).at[:2, :].set(W_e1)
    we2p = jnp.zeros((32, _S), f32).at[:, :2].set(W_e2)
    be2p = jnp.zeros((_S,), f32).at[:2].set(b_e2)

    out = pl.pallas_call(
        _fused_body,
        out_shape=jax.ShapeDtypeStruct((B, _S), f32),
    )(t2, x, W_season, b_season.reshape(1, -1), winx, wins,
      b_in.reshape(1, -1), W_self0, b0.reshape(1, -1), W_self1,
      b1.reshape(1, -1), W_out, b_out.reshape(1, -1), we1p,
      b_e1.reshape(1, -1), we2p, be2p.reshape(1, -1))
    return out


# speedup vs baseline: 686.5859x; 1.5592x over previous
import jax
import jax.numpy as jnp
import numpy as np
from jax.experimental import pallas as pl

_TWO_PI = 2.0 * np.pi
_S = 32
_H = 128


def _dot(a, b):
    return jnp.dot(a, b, preferred_element_type=jnp.float32)


def _fused_body(t_ref, x_ref, ws_ref, bs_ref, win_ref, bin_ref,
                w0_ref, b0_ref, w1_ref, b1_ref, wout_ref, bout_ref,
                we1_ref, be1_ref, we2_ref, be2_ref, out_ref):
    B = x_ref.shape[0]
    tv = t_ref[:]
    st = jnp.sin(_TWO_PI * tv)
    ct = jnp.cos(_TWO_PI * tv)
    s_emb = st * ws_ref[0:1, :] + ct * ws_ref[1:2, :] + bs_ref[:]
    win = win_ref[:]
    bias_eff = _dot(s_emb, win[_S:, :]) + bin_ref[:]
    x = x_ref[:]
    p = _dot(x, win[:_S, :]) + bias_eff
    h1 = jnp.tanh(_dot(p, w0_ref[:]) + b0_ref[:])
    wmean = jnp.mean(wout_ref[:], axis=1, keepdims=True)
    v = _dot(w1_ref[:], wmean)
    s = _dot(b1_ref[:], wmean) + jnp.mean(bout_ref[:])
    c = _dot(h1, v) + s
    e1 = jnp.tanh(_dot(x[:, 0:2], we1_ref[:]) + be1_ref[:])
    e2 = _dot(e1, we2_ref[:]) + be2_ref[:]
    lane = jax.lax.broadcasted_iota(jnp.int32, (B, _S), 1)
    out_ref[:] = (jnp.broadcast_to(c, (B, _S))
                  + jnp.where(lane == 0, e2[:, 0:1], 0.0)
                  + jnp.where(lane == 1, e2[:, 1:2], 0.0))


def kernel(t, x, W_season, b_season, W_in, b_in, W_self0, W_msg0, b0,
           W_self1, W_msg1, b1, W_out, b_out, W_e1, b_e1, W_e2, b_e2,
           edge_index):
    B = x.shape[0]
    return pl.pallas_call(
        _fused_body,
        out_shape=jax.ShapeDtypeStruct((B, _S), jnp.float32),
    )(t.reshape(1, 1), x, W_season, b_season.reshape(1, -1), W_in,
      b_in.reshape(1, -1), W_self0, b0.reshape(1, -1), W_self1,
      b1.reshape(1, -1), W_out, b_out.reshape(1, -1), W_e1,
      b_e1.reshape(1, -1), W_e2, b_e2.reshape(1, -1))
